# baseline (device time: 265571 ns/iter reference)
import jax
import jax.numpy as jnp
from jax import lax
from jax.experimental import pallas as pl
from jax.experimental.pallas import tpu as pltpu

M = 4096
N = 4096
KSH = 2048
NYH = N // 2
NC = 16
CM = M // NC
KB = 4
CK = KSH // KB


def kernel(A, B):
    c_init = jnp.zeros((M, N), jnp.bfloat16)

    def body(a_hbm, b_hbm, c_ref, out_ref, a_slots, b_stage, b16, p_buf,
             rx_buf, sa, sb, sx_send, sx_recv, sy_send, sy_recv, s_copy):
        my_x = lax.axis_index("x")
        my_y = lax.axis_index("y")
        col0 = my_y * NYH
        x_nbr = (1 - my_x, my_y)
        y_nbr = (my_x, 1 - my_y)

        def a_copy(i, slot):
            return pltpu.make_async_copy(
                a_hbm.at[pl.ds(i * CM, CM), :], a_slots.at[slot], sa.at[slot]
            )

        def b_copy(k, slot):
            return pltpu.make_async_copy(
                b_hbm.at[pl.ds(k * CK, CK), pl.ds(col0, NYH)],
                b_stage.at[slot],
                sb.at[slot],
            )

        barrier = pltpu.get_barrier_semaphore()
        for nbr in (x_nbr, y_nbr):
            pl.semaphore_signal(barrier, inc=1, device_id=nbr,
                                device_id_type=pl.DeviceIdType.MESH)
        pl.semaphore_wait(barrier, 2)

        b_copy(0, 0).start()
        a_copy(0, 0).start()
        for k in range(KB):
            if k + 1 < KB:
                b_copy(k + 1, (k + 1) % 2).start()
            b_copy(k, k % 2).wait()
            b16[pl.ds(k * CK, CK), :] = b_stage[k % 2].astype(jnp.bfloat16)

        x_descs = []
        for i in range(NC):
            slot = i % 2
            if i + 1 < NC:
                a_copy(i + 1, 1 - slot).start()
            a_copy(i, slot).wait()
            p = jnp.dot(a_slots[slot].astype(jnp.bfloat16), b16[...],
                        preferred_element_type=jnp.float32)
            p_buf[i] = p.astype(jnp.bfloat16)
            rdma_x = pltpu.make_async_remote_copy(
                src_ref=p_buf.at[i],
                dst_ref=rx_buf.at[i],
                send_sem=sx_send.at[i],
                recv_sem=sx_recv.at[i],
                device_id=x_nbr,
                device_id_type=pl.DeviceIdType.MESH,
            )
            rdma_x.start()
            x_descs.append(rdma_x)

        y_descs = []
        copies = []
        for i in range(NC):
            rows = pl.ds(i * CM, CM)
            x_descs[i].wait_recv()
            rx_buf[i] = p_buf[i] + rx_buf[i]
            rdma_y = pltpu.make_async_remote_copy(
                src_ref=rx_buf.at[i],
                dst_ref=out_ref.at[rows, pl.ds(col0, NYH)],
                send_sem=sy_send.at[i],
                recv_sem=sy_recv.at[i],
                device_id=y_nbr,
                device_id_type=pl.DeviceIdType.MESH,
            )
            rdma_y.start()
            y_descs.append(rdma_y)
            copy = pltpu.make_async_copy(
                rx_buf.at[i], out_ref.at[rows, pl.ds(col0, NYH)], s_copy.at[i]
            )
            copy.start()
            copies.append(copy)

        for i in range(NC):
            x_descs[i].wait_send()
            y_descs[i].wait_send()
            y_descs[i].wait_recv()
            copies[i].wait()

    return pl.pallas_call(
        body,
        out_shape=jax.ShapeDtypeStruct((M, N), jnp.bfloat16),
        in_specs=[
            pl.BlockSpec(memory_space=pl.ANY),
            pl.BlockSpec(memory_space=pl.ANY),
            pl.BlockSpec(memory_space=pl.ANY),
        ],
        out_specs=pl.BlockSpec(memory_space=pl.ANY),
        input_output_aliases={2: 0},
        scratch_shapes=[
            pltpu.VMEM((2, CM, KSH), jnp.float32),
            pltpu.VMEM((2, CK, NYH), jnp.float32),
            pltpu.VMEM((KSH, NYH), jnp.bfloat16),
            pltpu.VMEM((NC, CM, NYH), jnp.bfloat16),
            pltpu.VMEM((NC, CM, NYH), jnp.bfloat16),
            pltpu.SemaphoreType.DMA((2,)),
            pltpu.SemaphoreType.DMA((2,)),
            pltpu.SemaphoreType.DMA((NC,)),
            pltpu.SemaphoreType.DMA((NC,)),
            pltpu.SemaphoreType.DMA((NC,)),
            pltpu.SemaphoreType.DMA((NC,)),
            pltpu.SemaphoreType.DMA((NC,)),
        ],
        compiler_params=pltpu.CompilerParams(
            collective_id=0, vmem_limit_bytes=60 * 1024 * 1024
        ),
    )(A, B, c_init)


# device time: 254111 ns/iter; 1.0451x vs baseline; 1.0451x over previous
import jax
import jax.numpy as jnp
from jax import lax
from jax.experimental import pallas as pl
from jax.experimental.pallas import tpu as pltpu

M = 4096
N = 4096
KSH = 2048
NYH = N // 2
NC = 16
CM = M // NC
KB = 4
CK = KSH // KB


def kernel(A, B):
    def body(a_hbm, b_hbm, out_ref, a_slots, b_stage, b16, p_buf,
             rx_buf, sa, sb, sx_send, sx_recv, sy_send, sy_recv, s_copy):
        my_x = lax.axis_index("x")
        my_y = lax.axis_index("y")
        col0 = my_y * NYH
        x_nbr = (1 - my_x, my_y)
        y_nbr = (my_x, 1 - my_y)

        def a_copy(i, slot):
            return pltpu.make_async_copy(
                a_hbm.at[pl.ds(i * CM, CM), :], a_slots.at[slot], sa.at[slot]
            )

        def b_copy(k, slot):
            return pltpu.make_async_copy(
                b_hbm.at[pl.ds(k * CK, CK), pl.ds(col0, NYH)],
                b_stage.at[slot],
                sb.at[slot],
            )

        barrier = pltpu.get_barrier_semaphore()
        for nbr in (x_nbr, y_nbr):
            pl.semaphore_signal(barrier, inc=1, device_id=nbr,
                                device_id_type=pl.DeviceIdType.MESH)
        pl.semaphore_wait(barrier, 2)

        b_copy(0, 0).start()
        a_copy(0, 0).start()
        for k in range(KB):
            if k + 1 < KB:
                b_copy(k + 1, (k + 1) % 2).start()
            b_copy(k, k % 2).wait()
            b16[pl.ds(k * CK, CK), :] = b_stage[k % 2].astype(jnp.bfloat16)

        x_descs = []
        for i in range(NC):
            slot = i % 2
            if i + 1 < NC:
                a_copy(i + 1, 1 - slot).start()
            a_copy(i, slot).wait()
            p = jnp.dot(a_slots[slot].astype(jnp.bfloat16), b16[...],
                        preferred_element_type=jnp.float32)
            p_buf[i] = p.astype(jnp.bfloat16)
            rdma_x = pltpu.make_async_remote_copy(
                src_ref=p_buf.at[i],
                dst_ref=rx_buf.at[i],
                send_sem=sx_send.at[i],
                recv_sem=sx_recv.at[i],
                device_id=x_nbr,
                device_id_type=pl.DeviceIdType.MESH,
            )
            rdma_x.start()
            x_descs.append(rdma_x)

        y_descs = []
        copies = []
        for i in range(NC):
            rows = pl.ds(i * CM, CM)
            x_descs[i].wait_recv()
            rx_buf[i] = p_buf[i] + rx_buf[i]
            rdma_y = pltpu.make_async_remote_copy(
                src_ref=rx_buf.at[i],
                dst_ref=out_ref.at[rows, pl.ds(col0, NYH)],
                send_sem=sy_send.at[i],
                recv_sem=sy_recv.at[i],
                device_id=y_nbr,
                device_id_type=pl.DeviceIdType.MESH,
            )
            rdma_y.start()
            y_descs.append(rdma_y)
            copy = pltpu.make_async_copy(
                rx_buf.at[i], out_ref.at[rows, pl.ds(col0, NYH)], s_copy.at[i]
            )
            copy.start()
            copies.append(copy)

        for i in range(NC):
            x_descs[i].wait_send()
            y_descs[i].wait_send()
            y_descs[i].wait_recv()
            copies[i].wait()

    return pl.pallas_call(
        body,
        out_shape=jax.ShapeDtypeStruct((M, N), jnp.bfloat16),
        in_specs=[
            pl.BlockSpec(memory_space=pl.ANY),
            pl.BlockSpec(memory_space=pl.ANY),
        ],
        out_specs=pl.BlockSpec(memory_space=pl.ANY),
        scratch_shapes=[
            pltpu.VMEM((2, CM, KSH), jnp.float32),
            pltpu.VMEM((2, CK, NYH), jnp.float32),
            pltpu.VMEM((KSH, NYH), jnp.bfloat16),
            pltpu.VMEM((NC, CM, NYH), jnp.bfloat16),
            pltpu.VMEM((NC, CM, NYH), jnp.bfloat16),
            pltpu.SemaphoreType.DMA((2,)),
            pltpu.SemaphoreType.DMA((2,)),
            pltpu.SemaphoreType.DMA((NC,)),
            pltpu.SemaphoreType.DMA((NC,)),
            pltpu.SemaphoreType.DMA((NC,)),
            pltpu.SemaphoreType.DMA((NC,)),
            pltpu.SemaphoreType.DMA((NC,)),
        ],
        compiler_params=pltpu.CompilerParams(
            collective_id=0, vmem_limit_bytes=60 * 1024 * 1024
        ),
    )(A, B)


# device time: 253535 ns/iter; 1.0475x vs baseline; 1.0023x over previous
import jax
import jax.numpy as jnp
from jax import lax
from jax.experimental import pallas as pl
from jax.experimental.pallas import tpu as pltpu

M = 4096
N = 4096
KSH = 2048
NYH = N // 2
KB = 4
CK = KSH // KB

CHUNKS = [128, 128, 256] + [384] * 8 + [256, 128, 128]
OFFS = [sum(CHUNKS[:i]) for i in range(len(CHUNKS))]
NC = len(CHUNKS)
CMAX = max(CHUNKS)
assert sum(CHUNKS) == M


def kernel(A, B):
    def body(a_hbm, b_hbm, out_ref, a_slots, b_stage, b16, p_buf,
             rx_buf, sa, sb, sx_send, sx_recv, sy_send, sy_recv, s_copy):
        my_x = lax.axis_index("x")
        my_y = lax.axis_index("y")
        col0 = my_y * NYH
        x_nbr = (1 - my_x, my_y)
        y_nbr = (my_x, 1 - my_y)

        def a_copy(i, slot):
            return pltpu.make_async_copy(
                a_hbm.at[pl.ds(OFFS[i], CHUNKS[i]), :],
                a_slots.at[slot, pl.ds(0, CHUNKS[i])],
                sa.at[slot],
            )

        def b_copy(k, slot):
            return pltpu.make_async_copy(
                b_hbm.at[pl.ds(k * CK, CK), pl.ds(col0, NYH)],
                b_stage.at[slot],
                sb.at[slot],
            )

        def x_rdma(i):
            rows = pl.ds(OFFS[i], CHUNKS[i])
            return pltpu.make_async_remote_copy(
                src_ref=p_buf.at[rows],
                dst_ref=rx_buf.at[rows],
                send_sem=sx_send.at[i],
                recv_sem=sx_recv.at[i],
                device_id=x_nbr,
                device_id_type=pl.DeviceIdType.MESH,
            )

        barrier = pltpu.get_barrier_semaphore()
        for nbr in (x_nbr, y_nbr):
            pl.semaphore_signal(barrier, inc=1, device_id=nbr,
                                device_id_type=pl.DeviceIdType.MESH)
        pl.semaphore_wait(barrier, 2)

        b_copy(0, 0).start()
        a_copy(0, 0).start()

        a_copy(0, 0).wait()
        a_copy(1, 1).start()
        x_descs = []
        p0 = None
        for k in range(KB):
            if k + 1 < KB:
                b_copy(k + 1, (k + 1) % 2).start()
            b_copy(k, k % 2).wait()
            bk = b_stage[k % 2].astype(jnp.bfloat16)
            b16[pl.ds(k * CK, CK), :] = bk
            ak = a_slots[0, : CHUNKS[0], pl.ds(k * CK, CK)].astype(jnp.bfloat16)
            t = jnp.dot(ak, bk, preferred_element_type=jnp.float32)
            p0 = t if p0 is None else p0 + t
        p_buf[pl.ds(0, CHUNKS[0])] = p0.astype(jnp.bfloat16)
        d = x_rdma(0)
        d.start()
        x_descs.append(d)

        for i in range(1, NC):
            slot = i % 2
            if i + 1 < NC:
                a_copy(i + 1, 1 - slot).start()
            a_copy(i, slot).wait()
            p = jnp.dot(
                a_slots[slot, : CHUNKS[i]].astype(jnp.bfloat16), b16[...],
                preferred_element_type=jnp.float32,
            )
            p_buf[pl.ds(OFFS[i], CHUNKS[i])] = p.astype(jnp.bfloat16)
            d = x_rdma(i)
            d.start()
            x_descs.append(d)

        y_descs = []
        copies = []
        for i in range(NC):
            rows = pl.ds(OFFS[i], CHUNKS[i])
            x_descs[i].wait_recv()
            rx_buf[rows] = p_buf[rows] + rx_buf[rows]
            rdma_y = pltpu.make_async_remote_copy(
                src_ref=rx_buf.at[rows],
                dst_ref=out_ref.at[rows, pl.ds(col0, NYH)],
                send_sem=sy_send.at[i],
                recv_sem=sy_recv.at[i],
                device_id=y_nbr,
                device_id_type=pl.DeviceIdType.MESH,
            )
            rdma_y.start()
            y_descs.append(rdma_y)
            copy = pltpu.make_async_copy(
                rx_buf.at[rows], out_ref.at[rows, pl.ds(col0, NYH)], s_copy.at[i]
            )
            copy.start()
            copies.append(copy)

        for i in range(NC):
            x_descs[i].wait_send()
            y_descs[i].wait_send()
            y_descs[i].wait_recv()
            copies[i].wait()

    return pl.pallas_call(
        body,
        out_shape=jax.ShapeDtypeStruct((M, N), jnp.bfloat16),
        in_specs=[
            pl.BlockSpec(memory_space=pl.ANY),
            pl.BlockSpec(memory_space=pl.ANY),
        ],
        out_specs=pl.BlockSpec(memory_space=pl.ANY),
        scratch_shapes=[
            pltpu.VMEM((2, CMAX, KSH), jnp.float32),
            pltpu.VMEM((2, CK, NYH), jnp.float32),
            pltpu.VMEM((KSH, NYH), jnp.bfloat16),
            pltpu.VMEM((M, NYH), jnp.bfloat16),
            pltpu.VMEM((M, NYH), jnp.bfloat16),
            pltpu.SemaphoreType.DMA((2,)),
            pltpu.SemaphoreType.DMA((2,)),
            pltpu.SemaphoreType.DMA((NC,)),
            pltpu.SemaphoreType.DMA((NC,)),
            pltpu.SemaphoreType.DMA((NC,)),
            pltpu.SemaphoreType.DMA((NC,)),
            pltpu.SemaphoreType.DMA((NC,)),
        ],
        compiler_params=pltpu.CompilerParams(
            collective_id=0, vmem_limit_bytes=62 * 1024 * 1024
        ),
    )(A, B)
